# baseline (speedup 1.0000x reference)
import jax
import jax.numpy as jnp
from jax.experimental import pallas as pl

_ROWS = 2
_NSPLIT = (4, 2, 1)


def _heads_body(*refs):
    dn = (((1,), (1,)), ((), ()))
    n0, n1, n2 = _NSPLIT
    i = 0
    xs0 = refs[i:i + n0]; i += n0
    w0, b0 = refs[i:i + 2]; i += 2
    xs1 = refs[i:i + n1]; i += n1
    w1, b1 = refs[i:i + 2]; i += 2
    xs2 = refs[i:i + n2]; i += n2
    w2, b2 = refs[i:i + 2]; i += 2
    o0, o1, o2 = refs[i:i + 3]

    def head(xs, w, b, o):
        n = len(xs)
        blk = o.shape[1] // n
        for r in range(_ROWS):
            for j, x in enumerate(xs):
                o[r, j * blk:(j + 1) * blk] = jax.lax.dot_general(
                    x[r, 0], w[...], dn,
                    preferred_element_type=jnp.float32) + b[...]

    head(xs0, w0, b0, o0)
    head(xs1, w1, b1, o1)
    head(xs2, w2, b2, o2)


def kernel(feats_0, feats_1, feats_2, W0, b0, W1, b1, W2, b2):
    B = feats_0.shape[0]
    shapes = [feats_0.shape, feats_1.shape, feats_2.shape]
    xs = [jnp.transpose(f, (0, 2, 3, 1)).reshape(
              f.shape[0], f.shape[2] * f.shape[3], f.shape[1])
          for f in (feats_0, feats_1, feats_2)]
    ws = [W0, W1, W2]
    bs = [b.reshape(1, -1) for b in (b0, b1, b2)]
    out_dim = W0.shape[0]

    def full_spec(a):
        return pl.BlockSpec(a.shape, lambda b: (0,) * a.ndim)

    in_specs = []
    operands = []
    for x, w, bia, n in zip(xs, ws, bs, _NSPLIT):
        xv = x.reshape(B, n, x.shape[1] // n, x.shape[2])
        for j in range(n):
            operands.append(xv)
            in_specs.append(pl.BlockSpec(
                (_ROWS, 1, xv.shape[2], xv.shape[3]),
                lambda b, jj=j: (b, jj, 0, 0)))
        operands.extend([w, bia])
        in_specs.extend([full_spec(w), full_spec(bia)])

    out_shapes = [jax.ShapeDtypeStruct((B, x.shape[1], out_dim), jnp.float32)
                  for x in xs]
    out_specs = [pl.BlockSpec((_ROWS, x.shape[1], out_dim),
                              lambda b: (b, 0, 0))
                 for x in xs]

    outs = pl.pallas_call(
        _heads_body,
        grid=(B // _ROWS,),
        in_specs=in_specs,
        out_specs=out_specs,
        out_shape=out_shapes,
    )(*operands)

    return tuple(
        jnp.transpose(o.reshape(s[0], s[2], s[3], out_dim), (0, 3, 1, 2))
        for o, s in zip(outs, shapes)
    )
